# baseline (speedup 1.0000x reference)
import jax
import jax.numpy as jnp
from jax import lax
from jax.experimental import pallas as pl
from jax.experimental.pallas import tpu as pltpu

_NROW = 4096
_NEXP = 64
_RBLK = 512
_NR = _NROW // _RBLK
_NB = 4
_NCH = 1024
_NSPLIT = 4
_RSUB = _RBLK // _NSPLIT


def _gate_body(*refs):
    x_refs = refs[:_NSPLIT]
    wm_ref, wn_ref, bm_ref, bn_ref, out_ref, accm, accn = refs[_NSPLIT:]
    r = pl.program_id(1)

    @pl.when(r == 0)
    def _zero():
        accm[...] = jnp.zeros_like(accm)
        accn[...] = jnp.zeros_like(accn)

    dn = (((1,), (0,)), ((), ()))
    am = jnp.zeros_like(accm)
    an = jnp.zeros_like(accn)
    for i in range(_NSPLIT):
        xb = x_refs[i][0].astype(jnp.bfloat16)
        wm = wm_ref[...][:, i * _RSUB:(i + 1) * _RSUB]
        wn = wn_ref[...][:, i * _RSUB:(i + 1) * _RSUB]
        am += lax.dot_general(wm, xb, dn, preferred_element_type=jnp.float32)
        an += lax.dot_general(wn, xb, dn, preferred_element_type=jnp.float32)
    accm[...] += am
    accn[...] += an

    @pl.when(r == _NR - 1)
    def _finish():
        gm = accm[...] + bm_ref[...]
        gn = accn[...] + bn_ref[...]
        g = gm + jax.nn.softplus(gn)
        neg = jnp.float32(-jnp.inf)
        m1 = jnp.max(g, axis=0, keepdims=True)
        is1 = g == m1
        c1 = jnp.sum(is1.astype(jnp.float32), axis=0, keepdims=True)
        g2 = jnp.where(is1, neg, g)
        m2 = jnp.max(g2, axis=0, keepdims=True)
        is2 = g2 == m2
        c2 = jnp.sum(is2.astype(jnp.float32), axis=0, keepdims=True)
        g3 = jnp.where(is2, neg, g2)
        m3 = jnp.max(g3, axis=0, keepdims=True)
        third = jnp.where(c1 >= 3.0, m1, jnp.where(c1 + c2 >= 3.0, m2, m3))
        keep = g > third
        ex = jnp.where(keep, jnp.exp(g - m1), 0.0)
        probs = ex / jnp.sum(ex, axis=0, keepdims=True)
        out_ref[...] = probs.T


def _x_spec(i):
    return pl.BlockSpec((1, _RSUB, _NCH),
                        lambda b, r, i=i: (b, r * _NSPLIT + i, 0))


def kernel(x, W_main, b_main, W_noise, b_noise):
    xm = x.transpose(0, 1, 3, 2).reshape(_NB, _NROW, _NCH)
    Wm = W_main.astype(jnp.bfloat16)
    Wn = W_noise.astype(jnp.bfloat16)
    bm = b_main.reshape(_NEXP, 1)
    bn = b_noise.reshape(_NEXP, 1)
    return pl.pallas_call(
        _gate_body,
        grid=(_NB, _NR),
        in_specs=[_x_spec(i) for i in range(_NSPLIT)] + [
            pl.BlockSpec((_NEXP, _RBLK), lambda b, r: (0, r)),
            pl.BlockSpec((_NEXP, _RBLK), lambda b, r: (0, r)),
            pl.BlockSpec((_NEXP, 1), lambda b, r: (0, 0)),
            pl.BlockSpec((_NEXP, 1), lambda b, r: (0, 0)),
        ],
        out_specs=pl.BlockSpec((_NCH, _NEXP), lambda b, r: (b, 0)),
        out_shape=jax.ShapeDtypeStruct((_NB * _NCH, _NEXP), jnp.float32),
        scratch_shapes=[
            pltpu.VMEM((_NEXP, _NCH), jnp.float32),
            pltpu.VMEM((_NEXP, _NCH), jnp.float32),
        ],
    )(*([xm] * _NSPLIT), Wm, Wn, bm, bn)


# speedup vs baseline: 1.1066x; 1.1066x over previous
import jax
import jax.numpy as jnp
from jax import lax
from jax.experimental import pallas as pl
from jax.experimental.pallas import tpu as pltpu

_NROW = 4096
_NEXP = 64
_RBLK = 2048
_NR = _NROW // _RBLK
_NB = 4
_NCH = 1024
_NSPLIT = 4
_RSUB = _RBLK // _NSPLIT


def _gate_body(*refs):
    x_refs = refs[:_NSPLIT]
    wm_ref, wn_ref, bm_ref, bn_ref, out_ref, accm, accn = refs[_NSPLIT:]
    r = pl.program_id(1)

    @pl.when(r == 0)
    def _zero():
        accm[...] = jnp.zeros_like(accm)
        accn[...] = jnp.zeros_like(accn)

    dn = (((1,), (0,)), ((), ()))
    am = jnp.zeros_like(accm)
    an = jnp.zeros_like(accn)
    for i in range(_NSPLIT):
        xb = x_refs[i][0].astype(jnp.bfloat16)
        wm = wm_ref[...][:, i * _RSUB:(i + 1) * _RSUB]
        wn = wn_ref[...][:, i * _RSUB:(i + 1) * _RSUB]
        am += lax.dot_general(wm, xb, dn, preferred_element_type=jnp.float32)
        an += lax.dot_general(wn, xb, dn, preferred_element_type=jnp.float32)
    accm[...] += am
    accn[...] += an

    @pl.when(r == _NR - 1)
    def _finish():
        gm = accm[...] + bm_ref[...]
        gn = accn[...] + bn_ref[...]
        g = gm + jax.nn.softplus(gn)
        neg = jnp.float32(-jnp.inf)
        m1 = jnp.max(g, axis=0, keepdims=True)
        is1 = g == m1
        c1 = jnp.sum(is1.astype(jnp.float32), axis=0, keepdims=True)
        g2 = jnp.where(is1, neg, g)
        m2 = jnp.max(g2, axis=0, keepdims=True)
        is2 = g2 == m2
        c2 = jnp.sum(is2.astype(jnp.float32), axis=0, keepdims=True)
        g3 = jnp.where(is2, neg, g2)
        m3 = jnp.max(g3, axis=0, keepdims=True)
        third = jnp.where(c1 >= 3.0, m1, jnp.where(c1 + c2 >= 3.0, m2, m3))
        keep = g > third
        ex = jnp.where(keep, jnp.exp(g - m1), 0.0)
        probs = ex / jnp.sum(ex, axis=0, keepdims=True)
        out_ref[...] = probs.T


def _x_spec(i):
    return pl.BlockSpec((1, _RSUB, _NCH),
                        lambda b, r, i=i: (b, r * _NSPLIT + i, 0))


def kernel(x, W_main, b_main, W_noise, b_noise):
    xm = x.transpose(0, 1, 3, 2).reshape(_NB, _NROW, _NCH)
    Wm = W_main.astype(jnp.bfloat16)
    Wn = W_noise.astype(jnp.bfloat16)
    bm = b_main.reshape(_NEXP, 1)
    bn = b_noise.reshape(_NEXP, 1)
    return pl.pallas_call(
        _gate_body,
        grid=(_NB, _NR),
        in_specs=[_x_spec(i) for i in range(_NSPLIT)] + [
            pl.BlockSpec((_NEXP, _RBLK), lambda b, r: (0, r)),
            pl.BlockSpec((_NEXP, _RBLK), lambda b, r: (0, r)),
            pl.BlockSpec((_NEXP, 1), lambda b, r: (0, 0)),
            pl.BlockSpec((_NEXP, 1), lambda b, r: (0, 0)),
        ],
        out_specs=pl.BlockSpec((_NCH, _NEXP), lambda b, r: (b, 0)),
        out_shape=jax.ShapeDtypeStruct((_NB * _NCH, _NEXP), jnp.float32),
        scratch_shapes=[
            pltpu.VMEM((_NEXP, _NCH), jnp.float32),
            pltpu.VMEM((_NEXP, _NCH), jnp.float32),
        ],
    )(*([xm] * _NSPLIT), Wm, Wn, bm, bn)


# speedup vs baseline: 1.1146x; 1.0072x over previous
import jax
import jax.numpy as jnp
from jax import lax
from jax.experimental import pallas as pl
from jax.experimental.pallas import tpu as pltpu

_NROW = 4096
_NEXP = 64
_RBLK = 4096
_NR = _NROW // _RBLK
_NB = 4
_NCH = 1024
_NSPLIT = 4
_RSUB = _RBLK // _NSPLIT


def _gate_body(*refs):
    x_refs = refs[:_NSPLIT]
    wm_ref, wn_ref, bm_ref, bn_ref, out_ref, accm, accn = refs[_NSPLIT:]
    r = pl.program_id(1)

    @pl.when(r == 0)
    def _zero():
        accm[...] = jnp.zeros_like(accm)
        accn[...] = jnp.zeros_like(accn)

    dn = (((1,), (0,)), ((), ()))
    am = jnp.zeros_like(accm)
    an = jnp.zeros_like(accn)
    for i in range(_NSPLIT):
        xb = x_refs[i][0].astype(jnp.bfloat16)
        wm = wm_ref[...][:, i * _RSUB:(i + 1) * _RSUB]
        wn = wn_ref[...][:, i * _RSUB:(i + 1) * _RSUB]
        am += lax.dot_general(wm, xb, dn, preferred_element_type=jnp.float32)
        an += lax.dot_general(wn, xb, dn, preferred_element_type=jnp.float32)
    accm[...] += am
    accn[...] += an

    @pl.when(r == _NR - 1)
    def _finish():
        gm = accm[...] + bm_ref[...]
        gn = accn[...] + bn_ref[...]
        g = gm + jax.nn.softplus(gn)
        neg = jnp.float32(-jnp.inf)
        m1 = jnp.max(g, axis=0, keepdims=True)
        is1 = g == m1
        c1 = jnp.sum(is1.astype(jnp.float32), axis=0, keepdims=True)
        g2 = jnp.where(is1, neg, g)
        m2 = jnp.max(g2, axis=0, keepdims=True)
        is2 = g2 == m2
        c2 = jnp.sum(is2.astype(jnp.float32), axis=0, keepdims=True)
        g3 = jnp.where(is2, neg, g2)
        m3 = jnp.max(g3, axis=0, keepdims=True)
        third = jnp.where(c1 >= 3.0, m1, jnp.where(c1 + c2 >= 3.0, m2, m3))
        keep = g > third
        ex = jnp.where(keep, jnp.exp(g - m1), 0.0)
        probs = ex / jnp.sum(ex, axis=0, keepdims=True)
        out_ref[...] = probs.T


def _x_spec(i):
    return pl.BlockSpec((1, _RSUB, _NCH),
                        lambda b, r, i=i: (b, r * _NSPLIT + i, 0))


def kernel(x, W_main, b_main, W_noise, b_noise):
    xm = x.transpose(0, 1, 3, 2).reshape(_NB, _NROW, _NCH)
    Wm = W_main.astype(jnp.bfloat16)
    Wn = W_noise.astype(jnp.bfloat16)
    bm = b_main.reshape(_NEXP, 1)
    bn = b_noise.reshape(_NEXP, 1)
    return pl.pallas_call(
        _gate_body,
        grid=(_NB, _NR),
        in_specs=[_x_spec(i) for i in range(_NSPLIT)] + [
            pl.BlockSpec((_NEXP, _RBLK), lambda b, r: (0, r)),
            pl.BlockSpec((_NEXP, _RBLK), lambda b, r: (0, r)),
            pl.BlockSpec((_NEXP, 1), lambda b, r: (0, 0)),
            pl.BlockSpec((_NEXP, 1), lambda b, r: (0, 0)),
        ],
        out_specs=pl.BlockSpec((_NCH, _NEXP), lambda b, r: (b, 0)),
        out_shape=jax.ShapeDtypeStruct((_NB * _NCH, _NEXP), jnp.float32),
        scratch_shapes=[
            pltpu.VMEM((_NEXP, _NCH), jnp.float32),
            pltpu.VMEM((_NEXP, _NCH), jnp.float32),
        ],
    )(*([xm] * _NSPLIT), Wm, Wn, bm, bn)
